# baseline (speedup 1.0000x reference)
import jax
import jax.numpy as jnp
from jax import lax
from jax.experimental import pallas as pl
from jax.experimental.pallas import tpu as pltpu

_LAMBDA_BBOX = 5.0
_IOU_THR = 0.1
_NEG = -1e30
_BIG_I = 2 ** 30


def _make_body(n_real, n_batch, bpg):
  def _body(pch_ref, gt_ref, out_ref, *scr):
    iou_refs = scr[0:bpg]
    order_refs = scr[bpg:2 * bpg]
    acc_ref = scr[2 * bpg]
    gstep = pl.program_id(0)
    ngrid = pl.num_programs(0)
    G = iou_refs[0].shape[0]
    R, C = iou_refs[0].shape[1], iou_refs[0].shape[2]
    RR = R // 8

    flat_p = (lax.broadcasted_iota(jnp.int32, (R, C), 0) * C
              + lax.broadcasted_iota(jnp.int32, (R, C), 1))
    flat_s = (lax.broadcasted_iota(jnp.int32, (8, 128), 0) * 128
              + lax.broadcasted_iota(jnp.int32, (8, 128), 1))

    geom = []
    for bi in range(bpg):
      cx = pch_ref[bi, 0]
      cy = pch_ref[bi, 1]
      pw = pch_ref[bi, 2]
      ph = pch_ref[bi, 3]
      x1 = cx - pw / 2
      y1 = cy - ph / 2
      x2 = cx + pw / 2
      y2 = cy + ph / 2
      area_p = (x2 - x1) * (y2 - y1)
      geom.append((x1, y1, x2, y2, area_p))

    def gt_xyxy(bi, j):
      gx = gt_ref[bi, j, 0] / 512.0
      gy = gt_ref[bi, j, 1] / 512.0
      gw = gt_ref[bi, j, 2] / 512.0
      gh = gt_ref[bi, j, 3] / 512.0
      gx1 = gx - gw / 2
      gy1 = gy - gh / 2
      gx2 = gx + gw / 2
      gy2 = gy + gh / 2
      return gx, gy, gw, gh, gx1, gy1, gx2, gy2

    def iou_col(bi, j):
      _, _, _, _, gx1, gy1, gx2, gy2 = gt_xyxy(bi, j)
      x1, y1, x2, y2, area_p = geom[bi]
      ga = (gx2 - gx1) * (gy2 - gy1)
      ltx = jnp.maximum(x1, gx1)
      lty = jnp.maximum(y1, gy1)
      rbx = jnp.minimum(x2, gx2)
      rby = jnp.minimum(y2, gy2)
      iw = jnp.clip(rbx - ltx, 0.0, None)
      ih = jnp.clip(rby - lty, 0.0, None)
      inter = iw * ih
      union = area_p + ga - inter
      return inter / jnp.maximum(union, 1e-9)

    def l1(t, cms):
      out = list(cms)
      for u in range(4):
        j = t * 4 + u
        for bi in range(bpg):
          col = iou_col(bi, j)
          iou_refs[bi][pl.ds(j, 1)] = col[None]
          m = jnp.max(col, axis=(0, 1), keepdims=True)
          out[bi] = jnp.where(flat_s == j, m, out[bi])
      return tuple(out)

    cm0 = jnp.full((8, 128), _NEG, jnp.float32)
    cms = list(lax.fori_loop(0, G // 4, l1, (cm0,) * bpg))

    def fold8(x):
      return jnp.sum(x.reshape(RR, 8, C), axis=0)

    sub128 = lax.broadcasted_iota(jnp.int32, (128, 128), 0)
    lan128 = lax.broadcasted_iota(jnp.int32, (128, 128), 1)
    lane1 = lax.broadcasted_iota(jnp.int32, (1, 128), 1)
    ones_row = jnp.ones((1, 128), jnp.float32)
    ranks = []
    for bi in range(bpg):
      v_row = cms[bi][0:1, :]
      v_sub = jnp.transpose(jnp.broadcast_to(v_row, (128, 128)))
      beat = jnp.logical_or(
          v_sub > v_row,
          jnp.logical_and(v_sub == v_row, sub128 < lan128))
      beat_f = jnp.where(beat, 1.0, 0.0)
      ranks.append(lax.dot_general(ones_row, beat_f,
                                   (((1,), (0,)), ((), ())),
                                   preferred_element_type=jnp.float32))

    def extract_order(k):
      kf = k * jnp.float32(1.0)
      jjv = [jnp.min(jnp.where(ranks[bi] == kf, lane1, _BIG_I),
                     axis=(0, 1), keepdims=True) for bi in range(bpg)]
      for bi in range(bpg):
        order_refs[bi][k] = jjv[bi][0, 0]

    for k in range(8):
      extract_order(k)

    def l3(t, st):
      st = list(st)
      for u in range(4):
        k = t * 4 + u
        used = [st[2 * bi] for bi in range(bpg)]
        jj = [order_refs[bi][k] for bi in range(bpg)]
        col = [iou_refs[bi][pl.ds(jj[bi], 1)][0] for bi in range(bpg)]
        masked = [col[bi] - used[bi] for bi in range(bpg)]
        m2 = [jnp.max(masked[bi], axis=(0, 1), keepdims=True)
              for bi in range(bpg)]
        okv = [m2[bi] >= _IOU_THR for bi in range(bpg)]
        eq = [masked[bi] == m2[bi] for bi in range(bpg)]
        ii = [jnp.min(jnp.where(eq[bi], flat_p, _BIG_I), axis=(0, 1),
                      keepdims=True) for bi in range(bpg)]
        oh = [flat_p == ii[bi] for bi in range(bpg)]
        for bi in range(bpg):
          st[2 * bi] = jnp.where(
              jnp.logical_and(oh[bi], okv[bi]), 2.0, used[bi])
        for bi in range(bpg):
          ii_s = ii[bi][0, 0]
          r_i = ii_s // C
          c_i = ii_s - r_i * C
          hit = lane1 == c_i
          mk = jnp.where(jnp.logical_and(hit, okv[bi]), 1.0, 0.0)
          gx, gy, gw, gh, _, _, _, _ = gt_xyxy(bi, jj[bi])
          el = None
          for ch, g_ in ((0, gx), (1, gy), (2, gw), (3, gh)):
            rv = pch_ref[bi, ch, pl.ds(r_i, 1), :]
            d = jnp.abs(rv - g_)
            e = jnp.where(d < 1.0, 0.5 * d * d, d - 0.5)
            el = e if el is None else el + e
          st[2 * bi + 1] = st[2 * bi + 1] + mk * el
      for u in range(4):
        k2 = t * 4 + u + 8

        @pl.when(k2 < G)
        def _():
          extract_order(k2)
      return tuple(st)

    zv = jnp.zeros((1, C), jnp.float32)
    zu = jnp.zeros((R, C), jnp.float32)
    st0 = []
    for bi in range(bpg):
      st0 += [zu, zv]
    st = lax.fori_loop(0, G // 4, l3, tuple(st0))

    bacc_t = jnp.float32(0.0)
    cacc_t = jnp.float32(0.0)
    nm_t = jnp.float32(0.0)
    for bi in range(bpg):
      pc = pch_ref[bi, 4]
      matched = st[2 * bi] > 1.0
      logp = jnp.maximum(jnp.log(pc), -100.0)
      log1p_ = jnp.maximum(jnp.log(1.0 - pc), -100.0)
      conf = jnp.where(matched, -logp, -log1p_)
      bacc_t = bacc_t + jnp.sum(st[2 * bi + 1])
      cacc_t = cacc_t + jnp.sum(conf)
      nm_t = nm_t + jnp.sum(jnp.where(matched, 1.0, 0.0))

    @pl.when(gstep == 0)
    def _():
      acc_ref[0] = 0.0
      acc_ref[1] = 0.0
      acc_ref[2] = 0.0

    acc_ref[0] = acc_ref[0] + bacc_t
    acc_ref[1] = acc_ref[1] + cacc_t
    acc_ref[2] = acc_ref[2] + nm_t

    @pl.when(gstep == ngrid - 1)
    def _():
      tb = acc_ref[0]
      tcf = acc_ref[1]
      tm = acc_ref[2]
      nboxes = jnp.float32(n_batch * G)
      total_conf = tcf / jnp.float32(n_batch * n_real)
      has = tm > 0.0
      total_bbox = jnp.where(has, tb / jnp.maximum(tm, 1.0), 0.0)
      gap = jnp.where(has, (1.0 - tm / nboxes) * 2.0, 3.0)
      loss = _LAMBDA_BBOX * total_bbox + total_conf + gap
      rate = tm / nboxes
      o = jnp.where(flat_s == 0, loss,
          jnp.where(flat_s == 1, total_bbox,
          jnp.where(flat_s == 2, total_conf,
          jnp.where(flat_s == 3, gap,
          jnp.where(flat_s == 4, rate, 0.0)))))
      out_ref[...] = o

  return _body


def kernel(images, bboxes, preds):
  B, N, _ = preds.shape
  G = bboxes.shape[1]
  C = 128
  NPAD = ((N + 1023) // 1024) * 1024
  R = NPAD // C
  BPG = 4 if B % 4 == 0 else 1
  preds_p = jnp.pad(preds, ((0, 0), (0, NPAD - N), (0, 0)))
  pch = preds_p.transpose(0, 2, 1).reshape(B, 5, R, C)

  out = pl.pallas_call(
      _make_body(N, B, BPG),
      grid=(B // BPG,),
      in_specs=[
          pl.BlockSpec((BPG, 5, R, C), lambda i: (i, 0, 0, 0)),
          pl.BlockSpec((BPG, G, 4), lambda i: (i, 0, 0),
                       memory_space=pltpu.SMEM),
      ],
      out_specs=pl.BlockSpec((8, 128), lambda i: (0, 0)),
      out_shape=jax.ShapeDtypeStruct((8, 128), jnp.float32),
      scratch_shapes=(
          [pltpu.VMEM((G, R, C), jnp.float32) for _ in range(BPG)]
          + [pltpu.SMEM((128,), jnp.int32) for _ in range(BPG)]
          + [pltpu.SMEM((3,), jnp.float32)]
      ),
  )(pch, bboxes)
  return (out[0, 0], out[0, 1], out[0, 2], out[0, 3], out[0, 4])


# speedup vs baseline: 1.1735x; 1.1735x over previous
import jax
import jax.numpy as jnp
from jax import lax
from jax.experimental import pallas as pl
from jax.experimental.pallas import tpu as pltpu

_LAMBDA_BBOX = 5.0
_IOU_THR = 0.1
_NEG = -1e30
_BIG_I = 2 ** 30


def _make_body(n_real, n_batch, bpg):
  def _body(pch_ref, gt_ref, out_ref, *scr):
    iou_refs = scr[0:bpg]
    order_refs = scr[bpg:2 * bpg]
    acc_ref = scr[2 * bpg]
    gstep = pl.program_id(0)
    ngrid = pl.num_programs(0)
    G = iou_refs[0].shape[0]
    R, C = iou_refs[0].shape[1], iou_refs[0].shape[2]
    RR = R // 8

    flat_p = (lax.broadcasted_iota(jnp.int32, (R, C), 0) * C
              + lax.broadcasted_iota(jnp.int32, (R, C), 1))
    flat_s = (lax.broadcasted_iota(jnp.int32, (8, 128), 0) * 128
              + lax.broadcasted_iota(jnp.int32, (8, 128), 1))

    geom = []
    for bi in range(bpg):
      cx = pch_ref[bi, 0]
      cy = pch_ref[bi, 1]
      pw = pch_ref[bi, 2]
      ph = pch_ref[bi, 3]
      x1 = cx - pw / 2
      y1 = cy - ph / 2
      x2 = cx + pw / 2
      y2 = cy + ph / 2
      area_p = (x2 - x1) * (y2 - y1)
      geom.append((x1, y1, x2, y2, area_p))

    def gt_xyxy(bi, j):
      gx = gt_ref[bi, j, 0] / 512.0
      gy = gt_ref[bi, j, 1] / 512.0
      gw = gt_ref[bi, j, 2] / 512.0
      gh = gt_ref[bi, j, 3] / 512.0
      gx1 = gx - gw / 2
      gy1 = gy - gh / 2
      gx2 = gx + gw / 2
      gy2 = gy + gh / 2
      return gx, gy, gw, gh, gx1, gy1, gx2, gy2

    def iou_col(bi, j):
      _, _, _, _, gx1, gy1, gx2, gy2 = gt_xyxy(bi, j)
      x1, y1, x2, y2, area_p = geom[bi]
      ga = (gx2 - gx1) * (gy2 - gy1)
      ltx = jnp.maximum(x1, gx1)
      lty = jnp.maximum(y1, gy1)
      rbx = jnp.minimum(x2, gx2)
      rby = jnp.minimum(y2, gy2)
      iw = jnp.clip(rbx - ltx, 0.0, None)
      ih = jnp.clip(rby - lty, 0.0, None)
      inter = iw * ih
      union = area_p + ga - inter
      return inter / jnp.maximum(union, 1e-9)

    def l1(t, cms):
      out = list(cms)
      for u in range(4):
        j = t * 4 + u
        for bi in range(bpg):
          col = iou_col(bi, j)
          iou_refs[bi][pl.ds(j, 1)] = col[None]
          m = jnp.max(col, axis=(0, 1), keepdims=True)
          out[bi] = jnp.where(flat_s == j, m, out[bi])
      return tuple(out)

    cm0 = jnp.full((8, 128), _NEG, jnp.float32)
    cms = list(lax.fori_loop(0, G // 4, l1, (cm0,) * bpg))

    def fold8(x):
      return jnp.sum(x.reshape(RR, 8, C), axis=0)

    sub128 = lax.broadcasted_iota(jnp.int32, (128, 128), 0)
    lan128 = lax.broadcasted_iota(jnp.int32, (128, 128), 1)
    lane1 = lax.broadcasted_iota(jnp.int32, (1, 128), 1)
    ones_row = jnp.ones((1, 128), jnp.float32)
    ranks = []
    for bi in range(bpg):
      v_row = cms[bi][0:1, :]
      v_sub = jnp.transpose(jnp.broadcast_to(v_row, (128, 128)))
      beat = jnp.logical_or(
          v_sub > v_row,
          jnp.logical_and(v_sub == v_row, sub128 < lan128))
      beat_f = jnp.where(beat, 1.0, 0.0)
      ranks.append(lax.dot_general(ones_row, beat_f,
                                   (((1,), (0,)), ((), ())),
                                   preferred_element_type=jnp.float32))

    def l2(t, _):
      for u in range(4):
        k = t * 4 + u
        kf = k.astype(jnp.float32) if hasattr(k, 'astype') else float(k)
        jjv = [jnp.min(jnp.where(ranks[bi] == kf, lane1, _BIG_I),
                       axis=(0, 1), keepdims=True) for bi in range(bpg)]
        for bi in range(bpg):
          order_refs[bi][k] = jjv[bi][0, 0]
      return 0

    lax.fori_loop(0, G // 4, l2, 0)

    def l3(t, st):
      st = list(st)
      for u in range(4):
        k = t * 4 + u
        used = [st[2 * bi] for bi in range(bpg)]
        jj = [order_refs[bi][k] for bi in range(bpg)]
        col = [iou_refs[bi][pl.ds(jj[bi], 1)][0] for bi in range(bpg)]
        masked = [col[bi] - used[bi] for bi in range(bpg)]
        m2 = [jnp.max(masked[bi], axis=(0, 1), keepdims=True)
              for bi in range(bpg)]
        okv = [m2[bi] >= _IOU_THR for bi in range(bpg)]
        eq = [masked[bi] == m2[bi] for bi in range(bpg)]
        ii = [jnp.min(jnp.where(eq[bi], flat_p, _BIG_I), axis=(0, 1),
                      keepdims=True) for bi in range(bpg)]
        oh = [flat_p == ii[bi] for bi in range(bpg)]
        for bi in range(bpg):
          st[2 * bi] = jnp.where(
              jnp.logical_and(oh[bi], okv[bi]), 2.0, used[bi])
        for bi in range(bpg):
          ii_s = ii[bi][0, 0]
          r_i = ii_s // C
          c_i = ii_s - r_i * C
          hit = lane1 == c_i
          mk = jnp.where(jnp.logical_and(hit, okv[bi]), 1.0, 0.0)
          gx, gy, gw, gh, _, _, _, _ = gt_xyxy(bi, jj[bi])
          el = None
          for ch, g_ in ((0, gx), (1, gy), (2, gw), (3, gh)):
            rv = pch_ref[bi, ch, pl.ds(r_i, 1), :]
            d = jnp.abs(rv - g_)
            e = jnp.where(d < 1.0, 0.5 * d * d, d - 0.5)
            el = e if el is None else el + e
          st[2 * bi + 1] = st[2 * bi + 1] + mk * el
      return tuple(st)

    zv = jnp.zeros((1, C), jnp.float32)
    zu = jnp.zeros((R, C), jnp.float32)
    st0 = []
    for bi in range(bpg):
      st0 += [zu, zv]
    st = lax.fori_loop(0, G // 4, l3, tuple(st0))

    bacc_t = jnp.float32(0.0)
    cacc_t = jnp.float32(0.0)
    nm_t = jnp.float32(0.0)
    for bi in range(bpg):
      pc = pch_ref[bi, 4]
      matched = st[2 * bi] > 1.0
      logp = jnp.maximum(jnp.log(pc), -100.0)
      log1p_ = jnp.maximum(jnp.log(1.0 - pc), -100.0)
      conf = jnp.where(matched, -logp, -log1p_)
      bacc_t = bacc_t + jnp.sum(st[2 * bi + 1])
      cacc_t = cacc_t + jnp.sum(conf)
      nm_t = nm_t + jnp.sum(jnp.where(matched, 1.0, 0.0))

    @pl.when(gstep == 0)
    def _():
      acc_ref[0] = 0.0
      acc_ref[1] = 0.0
      acc_ref[2] = 0.0

    acc_ref[0] = acc_ref[0] + bacc_t
    acc_ref[1] = acc_ref[1] + cacc_t
    acc_ref[2] = acc_ref[2] + nm_t

    @pl.when(gstep == ngrid - 1)
    def _():
      tb = acc_ref[0]
      tcf = acc_ref[1]
      tm = acc_ref[2]
      nboxes = jnp.float32(n_batch * G)
      total_conf = tcf / jnp.float32(n_batch * n_real)
      has = tm > 0.0
      total_bbox = jnp.where(has, tb / jnp.maximum(tm, 1.0), 0.0)
      gap = jnp.where(has, (1.0 - tm / nboxes) * 2.0, 3.0)
      loss = _LAMBDA_BBOX * total_bbox + total_conf + gap
      rate = tm / nboxes
      o = jnp.where(flat_s == 0, loss,
          jnp.where(flat_s == 1, total_bbox,
          jnp.where(flat_s == 2, total_conf,
          jnp.where(flat_s == 3, gap,
          jnp.where(flat_s == 4, rate, 0.0)))))
      out_ref[...] = o

  return _body


def kernel(images, bboxes, preds):
  B, N, _ = preds.shape
  G = bboxes.shape[1]
  C = 128
  NPAD = ((N + 1023) // 1024) * 1024
  R = NPAD // C
  BPG = 4 if B % 4 == 0 else 1
  preds_p = jnp.pad(preds, ((0, 0), (0, NPAD - N), (0, 0)))
  pch = preds_p.transpose(0, 2, 1).reshape(B, 5, R, C)

  out = pl.pallas_call(
      _make_body(N, B, BPG),
      grid=(B // BPG,),
      in_specs=[
          pl.BlockSpec((BPG, 5, R, C), lambda i: (i, 0, 0, 0)),
          pl.BlockSpec((BPG, G, 4), lambda i: (i, 0, 0),
                       memory_space=pltpu.SMEM),
      ],
      out_specs=pl.BlockSpec((8, 128), lambda i: (0, 0)),
      out_shape=jax.ShapeDtypeStruct((8, 128), jnp.float32),
      scratch_shapes=(
          [pltpu.VMEM((G, R, C), jnp.float32) for _ in range(BPG)]
          + [pltpu.SMEM((128,), jnp.int32) for _ in range(BPG)]
          + [pltpu.SMEM((3,), jnp.float32)]
      ),
  )(pch, bboxes)
  return (out[0, 0], out[0, 1], out[0, 2], out[0, 3], out[0, 4])
